# baseline (speedup 1.0000x reference)
import functools

import jax
import jax.numpy as jnp
from jax import lax
from jax.experimental import pallas as pl
from jax.experimental.pallas import tpu as pltpu
from jax.experimental.pallas import tpu_sc as plsc

N_NODES = 10000
D = 128

NC = 2
NS = 16

HALF = 5120
CHUNK = 64
LCAP = 328
ACC_ROWS = 5632
ROWS_PER_TILE = ACC_ROWS // NS
DUMMY = HALF + 64
BIGPAD = 1 << 20

ICHUNK = 128
INCH = 160
E_PAD = NS * INCH * ICHUNK

_NBUF = 5
_LEAD = 3
_SLACK = _NBUF - _LEAD

_mesh = plsc.VectorSubcoreMesh(
    core_axis_name="c", subcore_axis_name="s", num_cores=NC, num_subcores=NS
)


def _fill_rows(buf, nrows, w16, value):
    def row(r, _):
        for c in range(w16):
            buf[r, pl.ds(c * 16, 16)] = jnp.full((16,), value, jnp.float32)
        return 0
    lax.fori_loop(0, nrows, row, 0)


def _prep_body(srci_hbm, dsti_hbm,
               degp_hbm, srcf_hbm, dstf_hbm, cnt_hbm,
               sstg, dstg, slist, dlist, ones, cbuf, dacc, sem):
    c = lax.axis_index("c")
    s = lax.axis_index("s")
    lo = c * HALF

    pltpu.sync_copy(srci_hbm.at[s], sstg)
    pltpu.sync_copy(dsti_hbm.at[s], dstg)

    _fill_rows(ones, CHUNK, 1, 0.0)
    base = s * ROWS_PER_TILE
    for k, (off, n) in enumerate([(0, 64), (64, 64), (128, 64), (192, 64),
                                  (256, 64), (320, 32)]):
        pltpu.sync_copy(ones.at[pl.ds(0, n)],
                        dacc.at[pl.ds(base + off, n)])
    _fill_rows(ones, CHUNK, 1, 1.0)

    def pre2(i, _):
        for g in range(4):
            slist[i, pl.ds(g * 16, 16)] = jnp.zeros((16,), jnp.int32)
            dlist[i, pl.ds(g * 16, 16)] = jnp.full((16,), DUMMY, jnp.int32)
        return 0
    lax.fori_loop(0, LCAP, pre2, 0)

    def frow(r, ovec):
        for g in range(8):
            srcv = sstg[r, pl.ds(g * 16, 16)]
            dstv = dstg[r, pl.ds(g * 16, 16)]
            m = (dstv >= lo) & (dstv < lo + HALF)
            dloc = dstv - lo
            incl = plsc.cumsum(jnp.where(m, 1, 0).astype(jnp.int32))
            pos = ovec + incl - 1
            plsc.store_scatter(slist, [pos >> 6, pos & 63], srcv, mask=m)
            plsc.store_scatter(dlist, [pos >> 6, pos & 63], dloc, mask=m)
            ovec = ovec + plsc.all_reduce_population_count(m)
        return ovec
    ovec = lax.fori_loop(0, INCH, frow, jnp.zeros((16,), jnp.int32))

    n = ovec[0]
    nch = (n + (CHUNK - 1)) // CHUNK

    plsc.subcore_barrier()

    def fire(j, _):
        pltpu.async_copy(ones, dacc.at[dlist.at[j]], sem, add=True)
        return 0
    lax.fori_loop(0, nch, fire, 0)
    def drain(j, _):
        pltpu.make_async_copy(ones, dacc.at[dlist.at[j]], sem).wait()
        return 0
    lax.fori_loop(0, nch, drain, 0)

    pltpu.sync_copy(slist, srcf_hbm.at[c, s])
    pltpu.sync_copy(dlist, dstf_hbm.at[c, s])
    cbuf[pl.ds(0, 16)] = ovec
    pltpu.sync_copy(cbuf, cnt_hbm.at[c, s])
    plsc.subcore_barrier()
    pltpu.sync_copy(dacc.at[pl.ds(base, ROWS_PER_TILE)],
                    degp_hbm.at[c, pl.ds(base, ROWS_PER_TILE)])


_prep_call = functools.partial(
    pl.kernel,
    _prep_body,
    out_type=(
        jax.ShapeDtypeStruct((NC, ACC_ROWS, 16), jnp.float32),
        jax.ShapeDtypeStruct((NC, NS, LCAP, CHUNK), jnp.int32),
        jax.ShapeDtypeStruct((NC, NS, LCAP, CHUNK), jnp.int32),
        jax.ShapeDtypeStruct((NC, NS, 16), jnp.int32),
    ),
    mesh=_mesh,
    compiler_params=pltpu.CompilerParams(use_tc_tiling_on_sc=False, needs_layout_passes=False),
    scratch_types=[
        pltpu.VMEM((INCH, ICHUNK), jnp.int32),
        pltpu.VMEM((INCH, ICHUNK), jnp.int32),
        pltpu.VMEM((LCAP, CHUNK), jnp.int32),
        pltpu.VMEM((LCAP, CHUNK), jnp.int32),
        pltpu.VMEM((CHUNK, 16), jnp.float32),
        pltpu.VMEM((16,), jnp.int32),
        pltpu.VMEM_SHARED((ACC_ROWS, 16), jnp.float32),
        pltpu.SemaphoreType.DMA,
    ],
)()


def _agg_body(gs_hbm, srcf_hbm, dstf_hbm, cnt_hbm, aggp_hbm,
              sidx, didx, cbuf, *rest):
    bufs = list(rest[:_NBUF])
    acc = rest[_NBUF]
    sg = list(rest[_NBUF + 1:_NBUF + 1 + _NBUF])
    ss = list(rest[_NBUF + 1 + _NBUF:_NBUF + 1 + 2 * _NBUF])
    c = lax.axis_index("c")
    s = lax.axis_index("s")

    pltpu.sync_copy(srcf_hbm.at[c, s], sidx)
    pltpu.sync_copy(dstf_hbm.at[c, s], didx)
    pltpu.sync_copy(cnt_hbm.at[c, s], cbuf)
    n = cbuf[pl.ds(0, 16)][0]
    nch = (n + (CHUNK - 1)) // CHUNK
    nch = jnp.maximum(((nch + _NBUF - 1) // _NBUF) * _NBUF, _NBUF)

    _fill_rows(bufs[0], CHUNK, D // 16, 0.0)
    base = s * ROWS_PER_TILE
    for off, m in [(0, 64), (64, 64), (128, 64), (192, 64),
                   (256, 64), (320, 32)]:
        pltpu.sync_copy(bufs[0].at[pl.ds(0, m)],
                        acc.at[pl.ds(base + off, m)])
    plsc.subcore_barrier()

    for b in range(_LEAD):
        pltpu.async_copy(gs_hbm.at[sidx.at[b]], bufs[b], sg[b])

    def step(i, _):
        g = i * _NBUF
        for b in range(_NBUF):
            j = g + b
            nb = (b + _LEAD) % _NBUF
            pltpu.make_async_copy(gs_hbm.at[sidx.at[j]], bufs[b], sg[b]).wait()
            @pl.when(j >= _SLACK)
            def _():
                pltpu.make_async_copy(
                    bufs[nb], acc.at[didx.at[j - _SLACK]], ss[nb]).wait()
            @pl.when(j + _LEAD < nch)
            def _():
                pltpu.async_copy(gs_hbm.at[sidx.at[j + _LEAD]], bufs[nb],
                                 sg[nb])
            pltpu.async_copy(bufs[b], acc.at[didx.at[j]], ss[b], add=True)
        return 0
    lax.fori_loop(0, nch // _NBUF, step, 0)

    pltpu.make_async_copy(bufs[_NBUF - 2], acc.at[didx.at[nch - 2]],
                          ss[_NBUF - 2]).wait()
    pltpu.make_async_copy(bufs[_NBUF - 1], acc.at[didx.at[nch - 1]],
                          ss[_NBUF - 1]).wait()
    plsc.subcore_barrier()

    pltpu.sync_copy(acc.at[pl.ds(base, ROWS_PER_TILE)],
                    aggp_hbm.at[c, pl.ds(base, ROWS_PER_TILE)])


_agg_call = functools.partial(
    pl.kernel,
    _agg_body,
    out_type=jax.ShapeDtypeStruct((NC, ACC_ROWS, D), jnp.float32),
    mesh=_mesh,
    compiler_params=pltpu.CompilerParams(use_tc_tiling_on_sc=False, needs_layout_passes=False),
    scratch_types=(
        [pltpu.VMEM((LCAP, CHUNK), jnp.int32),
         pltpu.VMEM((LCAP, CHUNK), jnp.int32),
         pltpu.VMEM((16,), jnp.int32)]
        + [pltpu.VMEM((CHUNK, D), jnp.float32) for _ in range(_NBUF)]
        + [pltpu.VMEM_SHARED((ACC_ROWS, D), jnp.float32)]
        + [pltpu.SemaphoreType.DMA for _ in range(2 * _NBUF)]
    ),
)()


def _dis_col(degp_ref):
    dcol = jnp.concatenate(
        [degp_ref[0, :HALF, 0:1],
         degp_ref[1, :N_NODES - HALF, 0:1]], axis=0)
    return lax.rsqrt(1.0 + dcol)


def _stitch(aggp_ref):
    return jnp.concatenate(
        [aggp_ref[0, :HALF, :], aggp_ref[1, :N_NODES - HALF, :]], axis=0)


def _mm_body(x_ref, w1_ref, h_ref):
    h_ref[...] = jnp.dot(x_ref[...], w1_ref[...],
                         preferred_element_type=jnp.float32)


def _scale_body(h_ref, degp_ref, gs_ref):
    gs_ref[...] = h_ref[...] * _dis_col(degp_ref)


def _mid_body(gs1_ref, aggp_ref, degp_ref, b1_ref, w2_ref, gs2_ref):
    dis = _dis_col(degp_ref)
    agg = _stitch(aggp_ref) + gs1_ref[...]
    h = jnp.maximum(agg * dis + b1_ref[...], 0.0)
    gs2_ref[...] = jnp.dot(
        h, w2_ref[...], preferred_element_type=jnp.float32) * dis


def _out_body(gs2_ref, aggp_ref, degp_ref, b2_ref, out_ref):
    dis = _dis_col(degp_ref)
    agg = _stitch(aggp_ref) + gs2_ref[...]
    out_ref[...] = agg * dis + b2_ref[...]


_mm_call = pl.pallas_call(
    _mm_body, out_shape=jax.ShapeDtypeStruct((N_NODES, D), jnp.float32))
_scale_call = pl.pallas_call(
    _scale_body, out_shape=jax.ShapeDtypeStruct((N_NODES, D), jnp.float32))
_mid_call = pl.pallas_call(
    _mid_body, out_shape=jax.ShapeDtypeStruct((N_NODES, D), jnp.float32))
_out_call = pl.pallas_call(
    _out_body, out_shape=jax.ShapeDtypeStruct((N_NODES, D), jnp.float32))


def kernel(x, edge_index, W1, b1, W2, b2):
    src = edge_index[0].astype(jnp.int32)
    dst = edge_index[1].astype(jnp.int32)
    pad = E_PAD - src.shape[0]
    src_p = jnp.concatenate(
        [src, jnp.zeros((pad,), jnp.int32)]).reshape(NS, INCH, ICHUNK)
    dst_p = jnp.concatenate(
        [dst, jnp.full((pad,), BIGPAD, jnp.int32)]).reshape(NS, INCH, ICHUNK)

    h1 = _mm_call(x, W1)
    degp, srcf, dstf, cnt = _prep_call(src_p, dst_p)
    gs1 = _scale_call(h1, degp)
    agg1 = _agg_call(gs1, srcf, dstf, cnt)
    gs2 = _mid_call(gs1, agg1, degp, b1, W2)
    agg2 = _agg_call(gs2, srcf, dstf, cnt)
    out = _out_call(gs2, agg2, degp, b2)
    return out


# speedup vs baseline: 1.0039x; 1.0039x over previous
import functools

import jax
import jax.numpy as jnp
from jax import lax
from jax.experimental import pallas as pl
from jax.experimental.pallas import tpu as pltpu
from jax.experimental.pallas import tpu_sc as plsc

N_NODES = 10000
D = 128

NC = 2
NS = 16

HALF = 5120
CHUNK = 64
LCAP = 328
ACC_ROWS = 5632
ROWS_PER_TILE = ACC_ROWS // NS
DUMMY = HALF + 64
BIGPAD = 1 << 20

ICHUNK = 128
INCH = 160
E_PAD = NS * INCH * ICHUNK

_NBUF = 5
_LEAD = 3
_SLACK = _NBUF - _LEAD

_mesh = plsc.VectorSubcoreMesh(
    core_axis_name="c", subcore_axis_name="s", num_cores=NC, num_subcores=NS
)


def _fill_rows(buf, nrows, w16, value):
    def row(r, _):
        for c in range(w16):
            buf[r, pl.ds(c * 16, 16)] = jnp.full((16,), value, jnp.float32)
        return 0
    lax.fori_loop(0, nrows, row, 0)


def _prep_body(srci_hbm, dsti_hbm,
               degp_hbm, srcf_hbm, dstf_hbm, cnt_hbm,
               sstg, dstg, slist, dlist, ones, cbuf, dacc, sem):
    c = lax.axis_index("c")
    s = lax.axis_index("s")
    lo = c * HALF

    pltpu.sync_copy(srci_hbm.at[s], sstg)
    pltpu.sync_copy(dsti_hbm.at[s], dstg)

    _fill_rows(ones, CHUNK, 1, 0.0)
    base = s * ROWS_PER_TILE
    for k, (off, n) in enumerate([(0, 64), (64, 64), (128, 64), (192, 64),
                                  (256, 64), (320, 32)]):
        pltpu.sync_copy(ones.at[pl.ds(0, n)],
                        dacc.at[pl.ds(base + off, n)])
    _fill_rows(ones, CHUNK, 1, 1.0)

    def pre2(i, _):
        for g in range(4):
            slist[i, pl.ds(g * 16, 16)] = jnp.zeros((16,), jnp.int32)
            dlist[i, pl.ds(g * 16, 16)] = jnp.full((16,), DUMMY, jnp.int32)
        return 0
    lax.fori_loop(0, LCAP, pre2, 0)

    def frow(r, ovec):
        for g in range(8):
            srcv = sstg[r, pl.ds(g * 16, 16)]
            dstv = dstg[r, pl.ds(g * 16, 16)]
            m = (dstv >= lo) & (dstv < lo + HALF)
            dloc = dstv - lo
            incl = plsc.cumsum(jnp.where(m, 1, 0).astype(jnp.int32))
            pos = ovec + incl - 1
            plsc.store_scatter(slist, [pos >> 6, pos & 63], srcv, mask=m)
            plsc.store_scatter(dlist, [pos >> 6, pos & 63], dloc, mask=m)
            ovec = ovec + plsc.all_reduce_population_count(m)
        return ovec
    ovec = lax.fori_loop(0, INCH, frow, jnp.zeros((16,), jnp.int32))

    n = ovec[0]
    nch = (n + (CHUNK - 1)) // CHUNK

    plsc.subcore_barrier()

    def fire(j, _):
        pltpu.async_copy(ones, dacc.at[dlist.at[j]], sem, add=True)
        return 0
    lax.fori_loop(0, nch, fire, 0)
    def drain(j, _):
        pltpu.make_async_copy(ones, dacc.at[dlist.at[j]], sem).wait()
        return 0
    lax.fori_loop(0, nch, drain, 0)

    pltpu.sync_copy(slist, srcf_hbm.at[c, s])
    pltpu.sync_copy(dlist, dstf_hbm.at[c, s])
    cbuf[pl.ds(0, 16)] = ovec
    pltpu.sync_copy(cbuf, cnt_hbm.at[c, s])
    plsc.subcore_barrier()
    pltpu.sync_copy(dacc.at[pl.ds(base, ROWS_PER_TILE)],
                    degp_hbm.at[c, pl.ds(base, ROWS_PER_TILE)])


_prep_call = functools.partial(
    pl.kernel,
    _prep_body,
    out_type=(
        jax.ShapeDtypeStruct((NC, ACC_ROWS, 16), jnp.float32),
        jax.ShapeDtypeStruct((NC, NS, LCAP, CHUNK), jnp.int32),
        jax.ShapeDtypeStruct((NC, NS, LCAP, CHUNK), jnp.int32),
        jax.ShapeDtypeStruct((NC, NS, 16), jnp.int32),
    ),
    mesh=_mesh,
    compiler_params=pltpu.CompilerParams(use_tc_tiling_on_sc=False, needs_layout_passes=False),
    scratch_types=[
        pltpu.VMEM((INCH, ICHUNK), jnp.int32),
        pltpu.VMEM((INCH, ICHUNK), jnp.int32),
        pltpu.VMEM((LCAP, CHUNK), jnp.int32),
        pltpu.VMEM((LCAP, CHUNK), jnp.int32),
        pltpu.VMEM((CHUNK, 16), jnp.float32),
        pltpu.VMEM((16,), jnp.int32),
        pltpu.VMEM_SHARED((ACC_ROWS, 16), jnp.float32),
        pltpu.SemaphoreType.DMA,
    ],
)()


def _agg_body(gs_hbm, srcf_hbm, dstf_hbm, cnt_hbm, aggp_hbm,
              sidx, didx, cbuf, *rest):
    bufs = list(rest[:_NBUF])
    acc = rest[_NBUF]
    sg = list(rest[_NBUF + 1:_NBUF + 1 + _NBUF])
    ss = list(rest[_NBUF + 1 + _NBUF:_NBUF + 1 + 2 * _NBUF])
    c = lax.axis_index("c")
    s = lax.axis_index("s")

    pltpu.sync_copy(srcf_hbm.at[c, s], sidx)
    pltpu.sync_copy(dstf_hbm.at[c, s], didx)
    pltpu.sync_copy(cnt_hbm.at[c, s], cbuf)
    n = cbuf[pl.ds(0, 16)][0]
    nch = (n + (CHUNK - 1)) // CHUNK
    nch = jnp.maximum(((nch + _NBUF - 1) // _NBUF) * _NBUF, _NBUF)

    _fill_rows(bufs[0], CHUNK, D // 16, 0.0)
    base = s * ROWS_PER_TILE
    for off, m in [(0, 64), (64, 64), (128, 64), (192, 64),
                   (256, 64), (320, 32)]:
        pltpu.sync_copy(bufs[0].at[pl.ds(0, m)],
                        acc.at[pl.ds(base + off, m)])
    plsc.subcore_barrier()

    for b in range(_LEAD):
        pltpu.async_copy(gs_hbm.at[sidx.at[b]], bufs[b], sg[b])

    def step(i, _):
        g = i * _NBUF
        for b in range(_NBUF):
            j = g + b
            nb = (b + _LEAD) % _NBUF
            pltpu.make_async_copy(gs_hbm.at[sidx.at[j]], bufs[b], sg[b]).wait()
            @pl.when(j >= _SLACK)
            def _():
                pltpu.make_async_copy(
                    bufs[nb], acc.at[didx.at[j - _SLACK]], ss[nb]).wait()
            @pl.when(j + _LEAD < nch)
            def _():
                pltpu.async_copy(gs_hbm.at[sidx.at[j + _LEAD]], bufs[nb],
                                 sg[nb])
            pltpu.async_copy(bufs[b], acc.at[didx.at[j]], ss[b], add=True)
        return 0
    lax.fori_loop(0, nch // _NBUF, step, 0)

    pltpu.make_async_copy(bufs[_NBUF - 2], acc.at[didx.at[nch - 2]],
                          ss[_NBUF - 2]).wait()
    pltpu.make_async_copy(bufs[_NBUF - 1], acc.at[didx.at[nch - 1]],
                          ss[_NBUF - 1]).wait()
    plsc.subcore_barrier()

    pltpu.sync_copy(acc.at[pl.ds(base, ROWS_PER_TILE)],
                    aggp_hbm.at[c, pl.ds(base, ROWS_PER_TILE)])


_agg_call = functools.partial(
    pl.kernel,
    _agg_body,
    out_type=jax.ShapeDtypeStruct((NC, ACC_ROWS, D), jnp.float32),
    mesh=_mesh,
    compiler_params=pltpu.CompilerParams(use_tc_tiling_on_sc=False, needs_layout_passes=False),
    scratch_types=(
        [pltpu.VMEM((LCAP, CHUNK), jnp.int32),
         pltpu.VMEM((LCAP, CHUNK), jnp.int32),
         pltpu.VMEM((16,), jnp.int32)]
        + [pltpu.VMEM((CHUNK, D), jnp.float32) for _ in range(_NBUF)]
        + [pltpu.VMEM_SHARED((ACC_ROWS, D), jnp.float32)]
        + [pltpu.SemaphoreType.DMA for _ in range(2 * _NBUF)]
    ),
)()


def _dis_col(degp_ref):
    dcol = jnp.concatenate(
        [degp_ref[0, :HALF, 0:1],
         degp_ref[1, :N_NODES - HALF, 0:1]], axis=0)
    return lax.rsqrt(1.0 + dcol)


def _stitch(aggp_ref):
    return jnp.concatenate(
        [aggp_ref[0, :HALF, :], aggp_ref[1, :N_NODES - HALF, :]], axis=0)


def _lin1_body(x_ref, w1_ref, degp_ref, gs_ref):
    dis = _dis_col(degp_ref)
    gs_ref[...] = jnp.dot(x_ref[...], w1_ref[...],
                          preferred_element_type=jnp.float32) * dis


def _mid_body(gs1_ref, aggp_ref, degp_ref, b1_ref, w2_ref, gs2_ref):
    dis = _dis_col(degp_ref)
    agg = _stitch(aggp_ref) + gs1_ref[...]
    h = jnp.maximum(agg * dis + b1_ref[...], 0.0)
    gs2_ref[...] = jnp.dot(
        h, w2_ref[...], preferred_element_type=jnp.float32) * dis


def _out_body(gs2_ref, aggp_ref, degp_ref, b2_ref, out_ref):
    dis = _dis_col(degp_ref)
    agg = _stitch(aggp_ref) + gs2_ref[...]
    out_ref[...] = agg * dis + b2_ref[...]


_lin1_call = pl.pallas_call(
    _lin1_body, out_shape=jax.ShapeDtypeStruct((N_NODES, D), jnp.float32))
_mid_call = pl.pallas_call(
    _mid_body, out_shape=jax.ShapeDtypeStruct((N_NODES, D), jnp.float32))
_out_call = pl.pallas_call(
    _out_body, out_shape=jax.ShapeDtypeStruct((N_NODES, D), jnp.float32))


def kernel(x, edge_index, W1, b1, W2, b2):
    src = edge_index[0].astype(jnp.int32)
    dst = edge_index[1].astype(jnp.int32)
    pad = E_PAD - src.shape[0]
    src_p = jnp.concatenate(
        [src, jnp.zeros((pad,), jnp.int32)]).reshape(NS, INCH, ICHUNK)
    dst_p = jnp.concatenate(
        [dst, jnp.full((pad,), BIGPAD, jnp.int32)]).reshape(NS, INCH, ICHUNK)

    degp, srcf, dstf, cnt = _prep_call(src_p, dst_p)
    gs1 = _lin1_call(x, W1, degp)
    agg1 = _agg_call(gs1, srcf, dstf, cnt)
    gs2 = _mid_call(gs1, agg1, degp, b1, W2)
    agg2 = _agg_call(gs2, srcf, dstf, cnt)
    out = _out_call(gs2, agg2, degp, b2)
    return out
